# Initial kernel scaffold; baseline (speedup 1.0000x reference)
import jax
import jax.numpy as jnp
from jax.experimental import pallas as pl


def kernel(im_input, gt_motion, gt_depth, m_kernel):
    raise NotImplementedError("write your pallas kernel here")


# speedup vs baseline: 38.7408x; 38.7408x over previous
import jax
import jax.numpy as jnp
from jax.experimental import pallas as pl
from jax.experimental.pallas import tpu as pltpu

_M_RANGE = 2
_K = 5
_N_CLASS = 25
_N_DEPTH = 2
_IM_CH = 3
_EPS = 1e-8


def _shift_y(z, d):
    if d == 0:
        return z
    h, w = z.shape
    zpad = jnp.zeros((abs(d), w), z.dtype)
    if d > 0:
        return jnp.concatenate([z[d:, :], zpad], axis=0)
    return jnp.concatenate([zpad, z[:h + d, :]], axis=0)


def _shift_x(z, d):
    if d == 0:
        return z
    h, w = z.shape
    zpad = jnp.zeros((h, abs(d)), z.dtype)
    if d > 0:
        return jnp.concatenate([z[:, d:], zpad], axis=1)
    return jnp.concatenate([zpad, z[:, :w + d]], axis=1)


def _axis_w(f_idx, frac, k):
    w = jnp.where(f_idx == float(k), 1.0 - frac, 0.0)
    if k >= 1:
        w = w + jnp.where(f_idx == float(k - 1), frac, 0.0)
    return w


def _gtnet_kernel(mot_ref, dep_ref, im_ref, pred_ref, mmask_ref, dmask_ref,
                  seg_ref):
    mx = mot_ref[0, 0]
    my = mot_ref[0, 1]
    fmx = jnp.floor(mx)
    fmy = jnp.floor(my)
    fx = mx - fmx
    fy = my - fmy
    ixf = fmx + float(_M_RANGE)
    iyf = fmy + float(_M_RANGE)

    dep = dep_ref[0, 0]
    d0 = (dep == 0).astype(jnp.float32)
    d1 = (dep == 1).astype(jnp.float32)
    dmask_ref[0, 0] = d0
    dmask_ref[0, 1] = d1

    im = [im_ref[0, ch] for ch in range(_IM_CH)]

    wy = [_axis_w(iyf, fy, r) for r in range(_K)]
    wyd = [wy[r] * d1 for r in range(_K)]

    zshape = mx.shape
    zero = jnp.zeros(zshape, jnp.float32)

    T = zero
    S1 = zero
    PT = [zero] * _IM_CH
    P1 = [zero] * _IM_CH

    for col in range(_K):
        wxc = _axis_w(ixf, fx, col)
        aT = zero
        aS = zero
        aPT = [zero] * _IM_CH
        aP1 = [zero] * _IM_CH
        for row in range(_K):
            t0 = wy[row] * wxc
            mmask_ref[0, _K * row + col] = t0
            t1 = wyd[row] * wxc
            dy = row - _M_RANGE
            aT = aT + _shift_y(t0, dy)
            aS = aS + _shift_y(t1, dy)
            for ch in range(_IM_CH):
                aPT[ch] = aPT[ch] + _shift_y(t0 * im[ch], dy)
                aP1[ch] = aP1[ch] + _shift_y(t1 * im[ch], dy)
        dx = col - _M_RANGE
        T = T + _shift_x(aT, dx)
        S1 = S1 + _shift_x(aS, dx)
        for ch in range(_IM_CH):
            PT[ch] = PT[ch] + _shift_x(aPT[ch], dx)
            P1[ch] = P1[ch] + _shift_x(aP1[ch], dx)

    Q = []
    for ch in range(_IM_CH):
        ys = zero
        for r in range(_K):
            ys = ys + _shift_y(im[ch], r - _M_RANGE)
        q = zero
        for c in range(_K):
            q = q + _shift_x(ys, c - _M_RANGE)
        Q.append(q)

    eps_tot = float(_N_CLASS) * _EPS
    total1 = S1 + eps_tot
    total2 = (T - S1) + eps_tot
    ratio1 = 1.0 / total1
    f1 = 1.0 - jnp.maximum(1.0 - ratio1, 0.0)
    sum1 = total1 * f1
    left2 = jnp.maximum(1.0 - sum1, 0.0)
    ratio2 = left2 / total2
    f2 = 1.0 - jnp.maximum(1.0 - ratio2, 0.0)

    for ch in range(_IM_CH):
        pred_ref[0, ch] = (f1 * P1[ch] + f2 * (PT[ch] - P1[ch])
                           + _EPS * (f1 + f2) * Q[ch])
    seg_ref[0, 0] = 1.0 - (sum1 + total2 * f2)


def kernel(im_input, gt_motion, gt_depth, m_kernel, *, interpret=False):
    B, _, H, W = gt_motion.shape
    im = im_input[:, -_IM_CH:, :, :]
    dep = gt_depth.astype(jnp.int32)

    out_shape = (
        jax.ShapeDtypeStruct((B, _IM_CH, H, W), jnp.float32),
        jax.ShapeDtypeStruct((B, _N_CLASS, H, W), jnp.float32),
        jax.ShapeDtypeStruct((B, _N_DEPTH, H, W), jnp.float32),
        jax.ShapeDtypeStruct((B, 1, H, W), jnp.float32),
    )

    def bspec(c):
        return pl.BlockSpec((1, c, H, W), lambda b: (b, 0, 0, 0))

    pred, m_mask, d_mask, seg = pl.pallas_call(
        _gtnet_kernel,
        grid=(B,),
        in_specs=[bspec(2), bspec(1), bspec(_IM_CH)],
        out_specs=(bspec(_IM_CH), bspec(_N_CLASS), bspec(_N_DEPTH), bspec(1)),
        out_shape=out_shape,
        compiler_params=pltpu.CompilerParams(
            dimension_semantics=("parallel",),
            vmem_limit_bytes=48 * 1024 * 1024,
        ),
        name="gtnet_fused",
        interpret=interpret,
    )(gt_motion, dep, im)
    return pred, m_mask, d_mask, seg
